# baseline (speedup 1.0000x reference)
import functools

import jax
import jax.numpy as jnp
from jax import lax
from jax.experimental import pallas as pl
from jax.experimental.pallas import tpu as pltpu

B, D, T = 8, 512, 1024
NQ, K, CD = 8, 1024, 256
TB = 512
PREC = lax.Precision.HIGHEST


def _mm(a, b):
    return lax.dot_general(a, b, (((1,), (0,)), ((), ())),
                           preferred_element_type=jnp.float32)


def _main_body(w_in, in_b, w_out, out_b, cbn, cbt_hi, cbt_mid, cbt_lo, z,
               zq_out, codes, lat, loss):
    bf = jnp.bfloat16
    r = z[0]
    lsum = jnp.float32(0.0)
    krow = lax.broadcasted_iota(jnp.int32, (1, K), 1)
    sel = jnp.concatenate([
        jnp.ones((1, K), jnp.float32),
        (krow // 32).astype(jnp.float32),
        (krow % 32).astype(jnp.float32),
    ], axis=0).astype(bf)
    for i in range(NQ):
        ze = _mm(w_in[i], r.astype(bf)) + in_b[:, i:i + 1]
        lat[0, i * CD:(i + 1) * CD, :] = ze
        nrm = jnp.sqrt(jnp.sum(ze * ze, axis=0, keepdims=True))
        enc_n = ze / jnp.maximum(nrm, 1e-12)
        s = _mm(cbn[i], enc_n.astype(bf))
        m = jnp.max(s, axis=0, keepdims=True)
        ohb = (s == m).astype(bf)
        ext = _mm(sel, ohb)
        cnt = ext[0:1, :]
        idxf = ext[1:2, :] * 32.0 + ext[2:3, :]

        def _no_tie(s=s, ohb=ohb, idxf=idxf):
            return idxf.astype(jnp.int32), ohb

        def _tie_fix(s=s, m=m):
            C = K // 4
            kio = lax.broadcasted_iota(jnp.int32, (C, TB), 0)
            idxe = jnp.full((1, TB), K, jnp.int32)
            for c in range(4):
                a = jnp.min(jnp.where(s[c * C:(c + 1) * C] == m,
                                      kio + c * C, K),
                            axis=0, keepdims=True)
                idxe = jnp.minimum(idxe, a)
            ohe = jnp.concatenate(
                [kio + c * C == idxe for c in range(4)], axis=0)
            return idxe, ohe.astype(bf)

        idx, oh = lax.cond(jnp.max(cnt) > 1.5, _tie_fix, _no_tie)
        codes[0, i:i + 1, :] = idx
        zq = (_mm(cbt_hi[i], oh) + _mm(cbt_mid[i], oh)) + _mm(cbt_lo[i], oh)
        dlt = ze - zq
        lsum = lsum + jnp.sum(dlt * dlt)
        zq_st = ze + (zq - ze)
        out = _mm(w_out[i], zq_st.astype(bf)) + out_b[:, i:i + 1]
        r = r - out
    zq_out[0] = z[0] - r
    first = pl.program_id(1) == 0
    part = jnp.reshape(lsum * jnp.float32(1.0 / (B * CD * T)), (1, 1, 1))

    @pl.when(first)
    def _():
        loss[...] = part

    @pl.when(jnp.logical_not(first))
    def _():
        loss[...] += part


@jax.jit
def _run(z, in_v, in_g3, out_v, out_g3, in_b3, out_b3, cb, cbt):
    bf = jnp.bfloat16
    n = jnp.sqrt(jnp.sum(in_v * in_v, axis=2, keepdims=True))
    w_in = (in_g3 * in_v / jnp.maximum(n, 1e-12)).astype(bf)
    n = jnp.sqrt(jnp.sum(out_v * out_v, axis=2, keepdims=True))
    w_out = (out_g3 * out_v / jnp.maximum(n, 1e-12)).astype(bf)
    n = jnp.sqrt(jnp.sum(cb * cb, axis=2, keepdims=True))
    cbn = (cb / jnp.maximum(n, 1e-12)).astype(bf)
    cbt_hi = cbt.astype(bf)
    r1 = cbt - cbt_hi.astype(jnp.float32)
    cbt_mid = r1.astype(bf)
    cbt_lo = (r1 - cbt_mid.astype(jnp.float32)).astype(bf)

    zq, codes, lat, loss = pl.pallas_call(
        _main_body,
        grid=(B, T // TB),
        in_specs=[
            pl.BlockSpec((NQ, CD, D), lambda b, t: (0, 0, 0)),
            pl.BlockSpec((CD, NQ), lambda b, t: (0, 0)),
            pl.BlockSpec((NQ, D, CD), lambda b, t: (0, 0, 0)),
            pl.BlockSpec((D, NQ), lambda b, t: (0, 0)),
            pl.BlockSpec((NQ, K, CD), lambda b, t: (0, 0, 0)),
            pl.BlockSpec((NQ, CD, K), lambda b, t: (0, 0, 0)),
            pl.BlockSpec((NQ, CD, K), lambda b, t: (0, 0, 0)),
            pl.BlockSpec((NQ, CD, K), lambda b, t: (0, 0, 0)),
            pl.BlockSpec((1, D, TB), lambda b, t: (b, 0, t)),
        ],
        out_specs=[
            pl.BlockSpec((1, D, TB), lambda b, t: (b, 0, t)),
            pl.BlockSpec((1, NQ, TB), lambda b, t: (b, 0, t)),
            pl.BlockSpec((1, NQ * CD, TB), lambda b, t: (b, 0, t)),
            pl.BlockSpec((1, 1, 1), lambda b, t: (b, 0, 0)),
        ],
        out_shape=[
            jax.ShapeDtypeStruct((B, D, T), jnp.float32),
            jax.ShapeDtypeStruct((B, NQ, T), jnp.int32),
            jax.ShapeDtypeStruct((B, NQ * CD, T), jnp.float32),
            jax.ShapeDtypeStruct((B, 1, 1), jnp.float32),
        ],
        compiler_params=pltpu.CompilerParams(
            dimension_semantics=("parallel", "arbitrary"),
        ),
    )(w_in, in_b3, w_out, out_b3, cbn, cbt_hi, cbt_mid, cbt_lo, z)
    return zq, codes, lat, loss


def kernel(z, in_v, in_g, in_b, out_v, out_g, out_b, codebooks):
    cbt = jnp.transpose(codebooks, (0, 2, 1))
    zq, codes, lat, loss = _run(
        z, in_v, in_g[..., None], out_v, out_g[..., None],
        jnp.transpose(in_b), jnp.transpose(out_b), codebooks, cbt)
    l = jnp.sum(loss[:, 0, 0])
    return zq, codes, lat, l, l


# speedup vs baseline: 1.4999x; 1.4999x over previous
import functools

import jax
import jax.numpy as jnp
from jax import lax
from jax.experimental import pallas as pl
from jax.experimental.pallas import tpu as pltpu

B, D, T = 8, 512, 1024
NQ, K, CD = 8, 1024, 256
TB = 1024
PREC = lax.Precision.HIGHEST


def _mm(a, b):
    return lax.dot_general(a, b, (((1,), (0,)), ((), ())),
                           preferred_element_type=jnp.float32)


def _main_body(w_in, in_b, w_out, out_b, cbn, cbt_hi, cbt_mid, cbt_lo, z,
               zq_out, codes, lat, loss):
    bf = jnp.bfloat16
    r = z[0]
    lsum = jnp.float32(0.0)
    kio = lax.broadcasted_iota(jnp.int32, (K, TB), 0)
    for i in range(NQ):
        ze = _mm(w_in[i], r.astype(bf)) + in_b[:, i:i + 1]
        lat[0, i * CD:(i + 1) * CD, :] = ze
        nrm = jnp.sqrt(jnp.sum(ze * ze, axis=0, keepdims=True))
        enc_n = ze / jnp.maximum(nrm, 1e-12)
        s = _mm(cbn[i], enc_n.astype(bf))
        m = jnp.max(s, axis=0, keepdims=True)
        idx = jnp.min(jnp.where(s == m, kio, K), axis=0, keepdims=True)
        oh = (kio == idx).astype(bf)
        codes[0, i:i + 1, :] = idx
        zq = (_mm(cbt_hi[i], oh) + _mm(cbt_mid[i], oh)) + _mm(cbt_lo[i], oh)
        dlt = ze - zq
        lsum = lsum + jnp.sum(dlt * dlt)
        zq_st = ze + (zq - ze)
        out = _mm(w_out[i], zq_st.astype(bf)) + out_b[:, i:i + 1]
        r = r - out
    zq_out[0] = z[0] - r
    first = pl.program_id(1) == 0
    part = jnp.reshape(lsum * jnp.float32(1.0 / (B * CD * T)), (1, 1, 1))

    @pl.when(first)
    def _():
        loss[...] = part

    @pl.when(jnp.logical_not(first))
    def _():
        loss[...] += part


@jax.jit
def _run(z, in_v, in_g3, out_v, out_g3, in_b3, out_b3, cb, cbt):
    bf = jnp.bfloat16
    n = jnp.sqrt(jnp.sum(in_v * in_v, axis=2, keepdims=True))
    w_in = (in_g3 * in_v / jnp.maximum(n, 1e-12)).astype(bf)
    n = jnp.sqrt(jnp.sum(out_v * out_v, axis=2, keepdims=True))
    w_out = (out_g3 * out_v / jnp.maximum(n, 1e-12)).astype(bf)
    n = jnp.sqrt(jnp.sum(cb * cb, axis=2, keepdims=True))
    cbn = (cb / jnp.maximum(n, 1e-12)).astype(bf)
    cbt_hi = cbt.astype(bf)
    r1 = cbt - cbt_hi.astype(jnp.float32)
    cbt_mid = r1.astype(bf)
    cbt_lo = (r1 - cbt_mid.astype(jnp.float32)).astype(bf)

    zq, codes, lat, loss = pl.pallas_call(
        _main_body,
        grid=(B, T // TB),
        in_specs=[
            pl.BlockSpec((NQ, CD, D), lambda b, t: (0, 0, 0)),
            pl.BlockSpec((CD, NQ), lambda b, t: (0, 0)),
            pl.BlockSpec((NQ, D, CD), lambda b, t: (0, 0, 0)),
            pl.BlockSpec((D, NQ), lambda b, t: (0, 0)),
            pl.BlockSpec((NQ, K, CD), lambda b, t: (0, 0, 0)),
            pl.BlockSpec((NQ, CD, K), lambda b, t: (0, 0, 0)),
            pl.BlockSpec((NQ, CD, K), lambda b, t: (0, 0, 0)),
            pl.BlockSpec((NQ, CD, K), lambda b, t: (0, 0, 0)),
            pl.BlockSpec((1, D, TB), lambda b, t: (b, 0, t)),
        ],
        out_specs=[
            pl.BlockSpec((1, D, TB), lambda b, t: (b, 0, t)),
            pl.BlockSpec((1, NQ, TB), lambda b, t: (b, 0, t)),
            pl.BlockSpec((1, NQ * CD, TB), lambda b, t: (b, 0, t)),
            pl.BlockSpec((1, 1, 1), lambda b, t: (b, 0, 0)),
        ],
        out_shape=[
            jax.ShapeDtypeStruct((B, D, T), jnp.float32),
            jax.ShapeDtypeStruct((B, NQ, T), jnp.int32),
            jax.ShapeDtypeStruct((B, NQ * CD, T), jnp.float32),
            jax.ShapeDtypeStruct((B, 1, 1), jnp.float32),
        ],
        compiler_params=pltpu.CompilerParams(
            dimension_semantics=("parallel", "arbitrary"),
        ),
    )(w_in, in_b3, w_out, out_b3, cbn, cbt_hi, cbt_mid, cbt_lo, z)
    return zq, codes, lat, loss


def kernel(z, in_v, in_g, in_b, out_v, out_g, out_b, codebooks):
    cbt = jnp.transpose(codebooks, (0, 2, 1))
    zq, codes, lat, loss = _run(
        z, in_v, in_g[..., None], out_v, out_g[..., None],
        jnp.transpose(in_b), jnp.transpose(out_b), codebooks, cbt)
    l = jnp.sum(loss[:, 0, 0])
    return zq, codes, lat, l, l
